# baseline (speedup 1.0000x reference)
import jax
import jax.numpy as jnp
from jax.experimental import pallas as pl
from jax.experimental.pallas import tpu as pltpu

B = 1024
F = 640
D = 256
NK = 64
TB = 128
NBT = B // TB
SQ = 16
NSQ = 64 // SQ


def _fused_body(pp_ref, w_ref, kings_ref, bias_ref, w1_ref, b1_ref,
                w2_ref, b2_ref, wout_ref, bout_ref, out_ref,
                xacc_ref, psum_ref):
    bt = pl.program_id(0)
    sq = pl.program_id(1)
    part = jnp.sum(pp_ref[...], axis=1)

    @pl.when(sq == 0)
    def _first_part():
        psum_ref[...] = part

    @pl.when(jnp.logical_and(sq > 0, sq < NSQ - 1))
    def _mid_part():
        psum_ref[...] = psum_ref[...] + part

    @pl.when(sq == NSQ - 1)
    def _last_part():
        ppf = (psum_ref[...] + part).astype(jnp.float32)
        kings = kings_ref[...]
        xt = jnp.zeros((TB, D), jnp.float32)
        for k in range(NK):
            wk = w_ref[k]
            m = (kings == k).astype(jnp.float32)
            msum = m[:, 0:1] + m[:, 1:2]
            z = jax.lax.dot_general(ppf, wk[:F, :],
                                    (((1,), (0,)), ((), ())),
                                    preferred_element_type=jnp.float32)
            xt = xt + msum * (z + wk[F:F + 1, :])
        xacc_ref[pl.ds(bt * TB, TB), :] = xt

    @pl.when(jnp.logical_and(bt == NBT - 1, sq == NSQ - 1))
    def _tail():
        x = xacc_ref[...] + bias_ref[...]
        x = jnp.clip(x, 0.0, 127.0)
        w1s = w1_ref[...][:, :D] + w1_ref[...][:, D:]
        h = jax.lax.dot_general(x, w1s, (((1,), (1,)), ((), ())),
                                preferred_element_type=jnp.float32)
        h = h + b1_ref[...]
        h = jnp.clip(jnp.floor(h * (1.0 / 64.0)), 0.0, 127.0)
        h = jax.lax.dot_general(h, w2_ref[...], (((1,), (1,)), ((), ())),
                                preferred_element_type=jnp.float32)
        h = h + b2_ref[...]
        h = jnp.clip(jnp.floor(h * (1.0 / 64.0)), 0.0, 127.0)
        v = jnp.sum(h * wout_ref[...]) + bout_ref[...]
        out_ref[...] = jnp.floor(v * (1.0 / 16.0))


def kernel(piece_positions, king_positions, input_weights, input_bias,
           w1, b1, w2, b2, w_out, b_out):
    out = pl.pallas_call(
        _fused_body,
        grid=(NBT, NSQ),
        in_specs=[
            pl.BlockSpec((TB, SQ, F), lambda i, j: (i, j, 0)),
            pl.BlockSpec((NK, F + 1, D), lambda i, j: (0, 0, 0)),
            pl.BlockSpec((TB, 2), lambda i, j: (i, 0)),
            pl.BlockSpec((1, D), lambda i, j: (0, 0)),
            pl.BlockSpec((32, 2 * D), lambda i, j: (0, 0)),
            pl.BlockSpec((1, 32), lambda i, j: (0, 0)),
            pl.BlockSpec((32, 32), lambda i, j: (0, 0)),
            pl.BlockSpec((1, 32), lambda i, j: (0, 0)),
            pl.BlockSpec((1, 32), lambda i, j: (0, 0)),
            pl.BlockSpec((1, 1), lambda i, j: (0, 0)),
        ],
        out_specs=pl.BlockSpec((1, 1), lambda i, j: (0, 0)),
        out_shape=jax.ShapeDtypeStruct((1, 1), jnp.float32),
        scratch_shapes=[
            pltpu.VMEM((B, D), jnp.float32),
            pltpu.VMEM((TB, F), jnp.int32),
        ],
        compiler_params=pltpu.CompilerParams(
            vmem_limit_bytes=63 * 1024 * 1024),
    )(
        piece_positions,
        input_weights,
        king_positions,
        input_bias.reshape(1, D),
        w1,
        b1.reshape(1, 32),
        w2,
        b2.reshape(1, 32),
        w_out.reshape(1, 32),
        b_out.reshape(1, 1),
    )
    return out.reshape((1,))


# speedup vs baseline: 1.1314x; 1.1314x over previous
import jax
import jax.numpy as jnp
from jax.experimental import pallas as pl
from jax.experimental.pallas import tpu as pltpu

B = 1024
F = 640
D = 256
NK = 64
TB = 128
NBT = B // TB
NSQ = 4
SQ = 64 // NSQ
KQ = NK // NSQ


def _fused_body(pp_ref, w_ref, kings_ref, bias_ref, w1_ref, b1_ref,
                w2_ref, b2_ref, wout_ref, bout_ref, out_ref,
                xacc_ref, psum_ref):
    bt = pl.program_id(0)
    sq = pl.program_id(1)
    par = jax.lax.rem(bt, 2)

    @pl.when(bt < NBT)
    def _reduce():
        part = jnp.sum(pp_ref[...], axis=1)

        @pl.when(sq == 0)
        def _first():
            psum_ref[pl.ds(par * TB, TB), :] = part

        @pl.when(sq > 0)
        def _rest():
            psum_ref[pl.ds(par * TB, TB), :] = (
                psum_ref[pl.ds(par * TB, TB), :] + part)

    @pl.when(bt > 0)
    def _matmul():
        ppf = psum_ref[pl.ds((1 - par) * TB, TB), :].astype(jnp.float32)
        kings = kings_ref[...]
        xt = jnp.zeros((TB, D), jnp.float32)
        for kk in range(KQ):
            k = sq * KQ + kk
            wk = w_ref[pl.ds(k, 1)][0]
            m = (kings == k).astype(jnp.float32)
            msum = m[:, 0:1] + m[:, 1:2]
            z = jax.lax.dot_general(ppf, wk[:F, :],
                                    (((1,), (0,)), ((), ())),
                                    preferred_element_type=jnp.float32)
            xt = xt + msum * (z + wk[F:F + 1, :])
        rows = pl.ds((bt - 1) * TB, TB)

        @pl.when(sq == 0)
        def _init_rows():
            xacc_ref[rows, :] = xt

        @pl.when(sq > 0)
        def _acc_rows():
            xacc_ref[rows, :] = xacc_ref[rows, :] + xt

    @pl.when(jnp.logical_and(bt == NBT, sq == NSQ - 1))
    def _tail():
        x = xacc_ref[...] + bias_ref[...]
        x = jnp.clip(x, 0.0, 127.0)
        w1s = w1_ref[...][:, :D] + w1_ref[...][:, D:]
        h = jax.lax.dot_general(x, w1s, (((1,), (1,)), ((), ())),
                                preferred_element_type=jnp.float32)
        h = h + b1_ref[...]
        h = jnp.clip(jnp.floor(h * (1.0 / 64.0)), 0.0, 127.0)
        h = jax.lax.dot_general(h, w2_ref[...], (((1,), (1,)), ((), ())),
                                preferred_element_type=jnp.float32)
        h = h + b2_ref[...]
        h = jnp.clip(jnp.floor(h * (1.0 / 64.0)), 0.0, 127.0)
        v = jnp.sum(h * wout_ref[...]) + bout_ref[...]
        out_ref[...] = jnp.floor(v * (1.0 / 16.0))


def kernel(piece_positions, king_positions, input_weights, input_bias,
           w1, b1, w2, b2, w_out, b_out):
    out = pl.pallas_call(
        _fused_body,
        grid=(NBT + 1, NSQ),
        in_specs=[
            pl.BlockSpec((TB, SQ, F),
                         lambda i, j: (jnp.minimum(i, NBT - 1), j, 0)),
            pl.BlockSpec((NK, F + 1, D), lambda i, j: (0, 0, 0)),
            pl.BlockSpec((TB, 2),
                         lambda i, j: (jnp.maximum(i - 1, 0), 0)),
            pl.BlockSpec((1, D), lambda i, j: (0, 0)),
            pl.BlockSpec((32, 2 * D), lambda i, j: (0, 0)),
            pl.BlockSpec((1, 32), lambda i, j: (0, 0)),
            pl.BlockSpec((32, 32), lambda i, j: (0, 0)),
            pl.BlockSpec((1, 32), lambda i, j: (0, 0)),
            pl.BlockSpec((1, 32), lambda i, j: (0, 0)),
            pl.BlockSpec((1, 1), lambda i, j: (0, 0)),
        ],
        out_specs=pl.BlockSpec((1, 1), lambda i, j: (0, 0)),
        out_shape=jax.ShapeDtypeStruct((1, 1), jnp.float32),
        scratch_shapes=[
            pltpu.VMEM((B, D), jnp.float32),
            pltpu.VMEM((2 * TB, F), jnp.int32),
        ],
        compiler_params=pltpu.CompilerParams(
            vmem_limit_bytes=63 * 1024 * 1024),
    )(
        piece_positions,
        input_weights,
        king_positions,
        input_bias.reshape(1, D),
        w1,
        b1.reshape(1, 32),
        w2,
        b2.reshape(1, 32),
        w_out.reshape(1, 32),
        b_out.reshape(1, 1),
    )
    return out.reshape((1,))


# speedup vs baseline: 1.2239x; 1.0818x over previous
import jax
import jax.numpy as jnp
from jax.experimental import pallas as pl
from jax.experimental.pallas import tpu as pltpu

B = 1024
F = 640
D = 256
NK = 64
KB = 16
PP_TILE = 128


def _ppsum_body(pp_ref, out_ref):
    out_ref[...] = jnp.sum(pp_ref[...], axis=1).astype(jnp.float32)


def _main_body(ppsum_ref, w_ref, kings_ref, bias_ref, w1_ref, b1_ref,
               w2_ref, b2_ref, wout_ref, bout_ref, out_ref, xacc_ref):
    step = pl.program_id(0)
    kings = kings_ref[...]
    pp = ppsum_ref[...]

    acc = jnp.zeros((B, D), jnp.float32)
    for j in range(KB):
        k = step * KB + j
        wk = w_ref[j]
        m = (kings == k).astype(jnp.float32)
        msum = m[:, 0:1] + m[:, 1:2]
        z = jax.lax.dot_general(pp, wk[:F, :],
                                (((1,), (0,)), ((), ())),
                                preferred_element_type=jnp.float32)
        acc = acc + msum * (z + wk[F:F + 1, :])

    @pl.when(step == 0)
    def _init():
        xacc_ref[...] = acc

    @pl.when(step > 0)
    def _acc():
        xacc_ref[...] = xacc_ref[...] + acc

    @pl.when(step == NK // KB - 1)
    def _tail():
        x = xacc_ref[...] + bias_ref[...]
        x = jnp.clip(x, 0.0, 127.0)
        w1s = w1_ref[...][:, :D] + w1_ref[...][:, D:]
        h = jax.lax.dot_general(x, w1s, (((1,), (1,)), ((), ())),
                                preferred_element_type=jnp.float32)
        h = h + b1_ref[...]
        h = jnp.clip(jnp.floor(h * (1.0 / 64.0)), 0.0, 127.0)
        h = jax.lax.dot_general(h, w2_ref[...], (((1,), (1,)), ((), ())),
                                preferred_element_type=jnp.float32)
        h = h + b2_ref[...]
        h = jnp.clip(jnp.floor(h * (1.0 / 64.0)), 0.0, 127.0)
        v = jnp.sum(h * wout_ref[...]) + bout_ref[...]
        out_ref[...] = jnp.floor(v * (1.0 / 16.0))


def kernel(piece_positions, king_positions, input_weights, input_bias,
           w1, b1, w2, b2, w_out, b_out):
    ppsum = pl.pallas_call(
        _ppsum_body,
        grid=(B // PP_TILE,),
        in_specs=[pl.BlockSpec((PP_TILE, 64, F), lambda i: (i, 0, 0))],
        out_specs=pl.BlockSpec((PP_TILE, F), lambda i: (i, 0)),
        out_shape=jax.ShapeDtypeStruct((B, F), jnp.float32),
    )(piece_positions)

    out = pl.pallas_call(
        _main_body,
        grid=(NK // KB,),
        in_specs=[
            pl.BlockSpec((B, F), lambda s: (0, 0)),
            pl.BlockSpec((KB, F + 1, D), lambda s: (s, 0, 0)),
            pl.BlockSpec((B, 2), lambda s: (0, 0)),
            pl.BlockSpec((1, D), lambda s: (0, 0)),
            pl.BlockSpec((32, 2 * D), lambda s: (0, 0)),
            pl.BlockSpec((1, 32), lambda s: (0, 0)),
            pl.BlockSpec((32, 32), lambda s: (0, 0)),
            pl.BlockSpec((1, 32), lambda s: (0, 0)),
            pl.BlockSpec((1, 32), lambda s: (0, 0)),
            pl.BlockSpec((1, 1), lambda s: (0, 0)),
        ],
        out_specs=pl.BlockSpec((1, 1), lambda s: (0, 0)),
        out_shape=jax.ShapeDtypeStruct((1, 1), jnp.float32),
        scratch_shapes=[pltpu.VMEM((B, D), jnp.float32)],
    )(
        ppsum,
        input_weights,
        king_positions,
        input_bias.reshape(1, D),
        w1,
        b1.reshape(1, 32),
        w2,
        b2.reshape(1, 32),
        w_out.reshape(1, 32),
        b_out.reshape(1, 1),
    )
    return out.reshape((1,))
